# Initial kernel scaffold; baseline (speedup 1.0000x reference)
import functools

import jax
import jax.numpy as jnp
from jax.experimental import pallas as pl
from jax.experimental.pallas import tpu as pltpu


def _dense_body(x_ref, gw_ref, w1_ref, w3_ref, w2_ref, out_ref, wfull_ref):
    e = pl.program_id(0)
    f = pl.program_id(1)
    n_e = pl.num_programs(0)

    @pl.when((e == 0) & (f == 0))
    def _router():
        logits = jnp.dot(x_ref[...], gw_ref[...].T,
                         preferred_element_type=jnp.float32)
        m = jnp.max(logits, axis=1, keepdims=True)
        p = jnp.exp(logits - m)
        p = p / jnp.sum(p, axis=1, keepdims=True)
        iota = jax.lax.broadcasted_iota(jnp.int32, p.shape, 1)
        m1 = jnp.max(p, axis=1, keepdims=True)
        i1 = jnp.min(jnp.where(p >= m1, iota, n_e), axis=1, keepdims=True)
        p2 = jnp.where(iota == i1, -1.0, p)
        m2 = jnp.max(p2, axis=1, keepdims=True)
        i2 = jnp.min(jnp.where(p2 >= m2, iota, n_e), axis=1, keepdims=True)
        wfull_ref[...] = (jnp.where(iota == i1, m1, 0.0)
                          + jnp.where(iota == i2, m2, 0.0))
        out_ref[...] = jnp.zeros_like(out_ref)

    xv = x_ref[...]
    a = jnp.dot(xv, w1_ref[0].T, preferred_element_type=jnp.float32)
    b = jnp.dot(xv, w3_ref[0].T, preferred_element_type=jnp.float32)
    h = (a * jax.nn.sigmoid(a)) * b
    y = jnp.dot(h, w2_ref[0].T, preferred_element_type=jnp.float32)
    iota = jax.lax.broadcasted_iota(jnp.int32, wfull_ref.shape, 1)
    we = jnp.sum(jnp.where(iota == e, wfull_ref[...], 0.0), axis=1,
                 keepdims=True)
    out_ref[...] += y * we


@functools.partial(jax.jit, static_argnames=("interpret",))
def _moe_dense(x, gate_w, w1, w2, w3, interpret=False):
    b, s, d = x.shape
    e, fdim = w1.shape[0], w1.shape[1]
    x2 = x.reshape(s, d)
    ft = min(512, fdim)
    nf = fdim // ft
    out = pl.pallas_call(
        _dense_body,
        grid=(e, nf),
        in_specs=[
            pl.BlockSpec((s, d), lambda ei, fi: (0, 0)),
            pl.BlockSpec((e, d), lambda ei, fi: (0, 0)),
            pl.BlockSpec((1, ft, d), lambda ei, fi: (ei, fi, 0)),
            pl.BlockSpec((1, ft, d), lambda ei, fi: (ei, fi, 0)),
            pl.BlockSpec((1, d, ft), lambda ei, fi: (ei, 0, fi)),
        ],
        out_specs=pl.BlockSpec((s, d), lambda ei, fi: (0, 0)),
        out_shape=jax.ShapeDtypeStruct((s, d), jnp.float32),
        scratch_shapes=[pltpu.VMEM((s, e), jnp.float32)],
        compiler_params=pltpu.CompilerParams(
            dimension_semantics=("arbitrary", "arbitrary")),
        interpret=interpret,
    )(x2, gate_w, w1, w3, w2)
    return out.reshape(b, s, d)


def kernel(x, gate_w, w1, w2, w3):
    return _moe_dense(x, gate_w, w1, w2, w3)


# speedup vs baseline: 1.3935x; 1.3935x over previous
import functools

import jax
import jax.numpy as jnp
from jax.experimental import pallas as pl
from jax.experimental.pallas import tpu as pltpu


def _dense_body(x_ref, gw_ref, w1_ref, w3_ref, w2_ref, out_ref, wfull_ref):
    e = pl.program_id(0)
    f = pl.program_id(1)
    n_e = pl.num_programs(0)

    @pl.when((e == 0) & (f == 0))
    def _router():
        logits = jnp.dot(x_ref[...], gw_ref[...].T,
                         preferred_element_type=jnp.float32)
        m = jnp.max(logits, axis=1, keepdims=True)
        p = jnp.exp(logits - m)
        p = p / jnp.sum(p, axis=1, keepdims=True)
        iota = jax.lax.broadcasted_iota(jnp.int32, p.shape, 1)
        m1 = jnp.max(p, axis=1, keepdims=True)
        i1 = jnp.min(jnp.where(p >= m1, iota, n_e), axis=1, keepdims=True)
        p2 = jnp.where(iota == i1, -1.0, p)
        m2 = jnp.max(p2, axis=1, keepdims=True)
        i2 = jnp.min(jnp.where(p2 >= m2, iota, n_e), axis=1, keepdims=True)
        wfull_ref[...] = (jnp.where(iota == i1, m1, 0.0)
                          + jnp.where(iota == i2, m2, 0.0))
        out_ref[...] = jnp.zeros_like(out_ref)

    xv = x_ref[...]
    a = jnp.dot(xv, w1_ref[0].T, preferred_element_type=jnp.float32)
    b = jnp.dot(xv, w3_ref[0].T, preferred_element_type=jnp.float32)
    h = (a * jax.nn.sigmoid(a)) * b
    y = jnp.dot(h, w2_ref[0].T, preferred_element_type=jnp.float32)
    iota = jax.lax.broadcasted_iota(jnp.int32, wfull_ref.shape, 1)
    we = jnp.sum(jnp.where(iota == e, wfull_ref[...], 0.0), axis=1,
                 keepdims=True)
    out_ref[...] += y * we


@functools.partial(jax.jit, static_argnames=("interpret",))
def _moe_dense(x, gate_w, w1, w2, w3, interpret=False):
    b, s, d = x.shape
    e, fdim = w1.shape[0], w1.shape[1]
    x2 = x.reshape(s, d)
    ft = min(256, fdim)
    nf = fdim // ft
    out = pl.pallas_call(
        _dense_body,
        grid=(e, nf),
        in_specs=[
            pl.BlockSpec((s, d), lambda ei, fi: (0, 0)),
            pl.BlockSpec((e, d), lambda ei, fi: (0, 0)),
            pl.BlockSpec((1, ft, d), lambda ei, fi: (ei, fi, 0)),
            pl.BlockSpec((1, ft, d), lambda ei, fi: (ei, fi, 0)),
            pl.BlockSpec((1, d, ft), lambda ei, fi: (ei, 0, fi)),
        ],
        out_specs=pl.BlockSpec((s, d), lambda ei, fi: (0, 0)),
        out_shape=jax.ShapeDtypeStruct((s, d), jnp.float32),
        scratch_shapes=[pltpu.VMEM((s, e), jnp.float32)],
        compiler_params=pltpu.CompilerParams(
            dimension_semantics=("arbitrary", "arbitrary")),
        interpret=interpret,
    )(x2, gate_w, w1, w3, w2)
    return out.reshape(b, s, d)


def kernel(x, gate_w, w1, w2, w3):
    return _moe_dense(x, gate_w, w1, w2, w3)


# speedup vs baseline: 1.4230x; 1.0212x over previous
import functools

import jax
import jax.numpy as jnp
from jax import lax
from jax.experimental import pallas as pl
from jax.experimental.pallas import tpu as pltpu
from jax.experimental.pallas import tpu_sc as plsc

S = 2048
D = 2048
E = 8
K = 2
F = 4096
P = S * K
TM = 256
NB = P // TM + E
NPAD = NB * TM
NW = 32
LANES = 16


def _router_body(x_ref, gw_ref, dest_ref, topw_ref, meta_ref):
    xv = x_ref[...]
    logits = jnp.dot(xv, gw_ref[...].T, preferred_element_type=jnp.float32)
    m = jnp.max(logits, axis=1, keepdims=True)
    p = jnp.exp(logits - m)
    p = p / jnp.sum(p, axis=1, keepdims=True)
    iota8 = lax.broadcasted_iota(jnp.int32, (S, E), 1)
    m1 = jnp.max(p, axis=1, keepdims=True)
    i1 = jnp.min(jnp.where(p >= m1, iota8, E), axis=1, keepdims=True)
    p2 = jnp.where(iota8 == i1, -1.0, p)
    m2 = jnp.max(p2, axis=1, keepdims=True)
    i2 = jnp.min(jnp.where(p2 >= m2, iota8, E), axis=1, keepdims=True)

    sel = jnp.logical_or(iota8 == i1, iota8 == i2).astype(jnp.float32)
    r_io = lax.broadcasted_iota(jnp.int32, (S, S), 0)
    c_io = lax.broadcasted_iota(jnp.int32, (S, S), 1)
    tril = (c_io < r_io).astype(jnp.float32)
    rank = jnp.dot(tril, sel, preferred_element_type=jnp.float32)
    cnt_f = rank[S - 1:S, :] + sel[S - 1:S, :]
    cnt = cnt_f.astype(jnp.int32)
    pc = ((cnt + (TM - 1)) // TM) * TM
    u8r = lax.broadcasted_iota(jnp.int32, (E, E), 0)
    u8c = lax.broadcasted_iota(jnp.int32, (E, E), 1)
    ltmask = (u8r < u8c).astype(jnp.float32)
    pstart_f = jnp.dot(pc.astype(jnp.float32), ltmask,
                       preferred_element_type=jnp.float32)
    pstart = pstart_f.astype(jnp.int32)

    base = rank.astype(jnp.int32) + pstart
    d1 = jnp.sum(jnp.where(iota8 == i1, base, 0), axis=1, keepdims=True)
    d2 = jnp.sum(jnp.where(iota8 == i2, base, 0), axis=1, keepdims=True)
    dest_ref[:, 0:1] = d1
    dest_ref[:, 1:2] = d2
    topw_ref[:, 0:1] = m1
    topw_ref[:, 1:2] = m2

    total = pstart[0:1, E - 1:E] + pc[0:1, E - 1:E]
    emax = jnp.max(jnp.where(cnt > 0, iota8[0:1, :], 0), axis=1,
                   keepdims=True)
    bb = lax.broadcasted_iota(jnp.int32, (1, 128), 1) * TM
    bexp = jnp.zeros((1, 128), jnp.int32)
    for ei in range(E):
        ps_e = pstart[0:1, ei:ei + 1]
        bexp = bexp + (bb >= ps_e).astype(jnp.int32)
    bexp = jnp.minimum(jnp.maximum(bexp - 1, 0), emax)
    bvalid = (bb < total).astype(jnp.int32)
    meta_ref[0:1, :] = bexp
    meta_ref[1:2, :] = bvalid


@jax.jit
def _router(x2, gate_w):
    return pl.pallas_call(
        _router_body,
        out_shape=[
            jax.ShapeDtypeStruct((S, K), jnp.int32),
            jax.ShapeDtypeStruct((S, K), jnp.float32),
            jax.ShapeDtypeStruct((2, 128), jnp.int32),
        ],
    )(x2, gate_w)


def _gather_body(x_ref, dest_ref, xg_ref, dloc, table, rows, sem):
    wid = lax.axis_index("s") * 2 + lax.axis_index("c")
    pltpu.sync_copy(dest_ref, dloc)

    def zero_blk(i, c):
        table[pl.ds(i * LANES, LANES)] = jnp.zeros((LANES,), jnp.int32)
        return c
    lax.fori_loop(0, NPAD // LANES, zero_blk, 0)

    def scat_blk(i, c):
        idx = dloc[pl.ds(i * LANES, LANES)]
        tok = (lax.iota(jnp.int32, LANES) + i * LANES) >> 1
        plsc.store_scatter(table, [idx], tok)
        return c
    lax.fori_loop(0, P // LANES, scat_blk, 0)

    spw = NPAD // NW
    base = wid * spw

    def gat_blk(i, c):
        sb = base + i * LANES
        pltpu.async_copy(x_ref.at[table.at[pl.ds(sb, LANES)]], rows, sem).wait()
        pltpu.sync_copy(rows, xg_ref.at[pl.ds(sb, LANES)])
        return c
    lax.fori_loop(0, spw // LANES, gat_blk, 0)


@jax.jit
def _sc_gather(x2, destf):
    mesh = plsc.VectorSubcoreMesh(core_axis_name="c", subcore_axis_name="s")
    kern = pl.kernel(
        _gather_body,
        out_type=jax.ShapeDtypeStruct((NPAD, D), jnp.float32),
        mesh=mesh,
        compiler_params=pltpu.CompilerParams(needs_layout_passes=False),
        scratch_types=[
            pltpu.VMEM((P,), jnp.int32),
            pltpu.VMEM((NPAD,), jnp.int32),
            pltpu.VMEM((LANES, D), jnp.float32),
            pltpu.SemaphoreType.DMA,
        ],
    )
    return kern(x2, destf)


FT1 = 1024
NF1 = F // FT1


def _ffn_in_body(bexp_s, bval_s, xg_ref, w1_ref, w3_ref, h_ref):
    b = pl.program_id(1)

    @pl.when(bval_s[b] == 1)
    def _():
        xv = xg_ref[...]
        a = jnp.dot(xv, w1_ref[0].T, preferred_element_type=jnp.float32)
        g = jnp.dot(xv, w3_ref[0].T, preferred_element_type=jnp.float32)
        h_ref[...] = (a * jax.nn.sigmoid(a)) * g


@jax.jit
def _ffn_in(xg, w1, w3, bexp, bval):
    grid_spec = pltpu.PrefetchScalarGridSpec(
        num_scalar_prefetch=2,
        grid=(NF1, NB),
        in_specs=[
            pl.BlockSpec((TM, D), lambda f, b, be, bv: (b, 0)),
            pl.BlockSpec((1, FT1, D), lambda f, b, be, bv: (be[b], f, 0)),
            pl.BlockSpec((1, FT1, D), lambda f, b, be, bv: (be[b], f, 0)),
        ],
        out_specs=pl.BlockSpec((TM, FT1), lambda f, b, be, bv: (b, f)),
    )
    return pl.pallas_call(
        _ffn_in_body,
        grid_spec=grid_spec,
        out_shape=jax.ShapeDtypeStruct((NPAD, F), jnp.float32),
        compiler_params=pltpu.CompilerParams(
            dimension_semantics=("arbitrary", "arbitrary")),
    )(bexp, bval, xg, w1, w3)


FT2 = 2048
NF2 = F // FT2


def _ffn_out_body(bexp_s, bval_s, h_ref, w2_ref, y_ref):
    b = pl.program_id(0)
    f = pl.program_id(1)

    @pl.when(bval_s[b] == 1)
    def _():
        y = jnp.dot(h_ref[...], w2_ref[0].T, preferred_element_type=jnp.float32)

        @pl.when(f == 0)
        def _():
            y_ref[...] = y

        @pl.when(f > 0)
        def _():
            y_ref[...] += y


@jax.jit
def _ffn_out(h, w2, bexp, bval):
    def w2_map(b, f, be, bv):
        fs = jnp.where(b % 2 == 0, f, NF2 - 1 - f)
        return (be[b], 0, fs)

    def h_map(b, f, be, bv):
        fs = jnp.where(b % 2 == 0, f, NF2 - 1 - f)
        return (b, fs)

    grid_spec = pltpu.PrefetchScalarGridSpec(
        num_scalar_prefetch=2,
        grid=(NB, NF2),
        in_specs=[
            pl.BlockSpec((TM, FT2), h_map),
            pl.BlockSpec((1, D, FT2), w2_map),
        ],
        out_specs=pl.BlockSpec((TM, D), lambda b, f, be, bv: (b, 0)),
    )
    return pl.pallas_call(
        _ffn_out_body,
        grid_spec=grid_spec,
        out_shape=jax.ShapeDtypeStruct((NPAD, D), jnp.float32),
        compiler_params=pltpu.CompilerParams(
            dimension_semantics=("arbitrary", "arbitrary")),
    )(bexp, bval, h, w2)


def _combine_body(y_ref, dest_ref, w_ref, out_ref, didx, wloc, rows, acc, sem):
    wid = lax.axis_index("s") * 2 + lax.axis_index("c")
    tpw = S // NW
    tpc = LANES // K

    def chunk(i, c):
        pb = wid * tpw * K + i * LANES
        pltpu.sync_copy(dest_ref.at[pl.ds(pb, LANES)], didx)
        pltpu.async_copy(y_ref.at[didx], rows, sem).wait()
        pltpu.sync_copy(w_ref.at[pl.ds(pb, LANES)], wloc)
        wv = wloc[...]
        wj = [wv[j] for j in range(LANES)]

        def col(cc, c2):
            sl = pl.ds(cc * LANES, LANES)
            for j in range(tpc):
                acc[j, sl] = rows[2 * j, sl] * wj[2 * j] \
                    + rows[2 * j + 1, sl] * wj[2 * j + 1]
            return c2
        lax.fori_loop(0, D // LANES, col, 0)
        pltpu.sync_copy(acc, out_ref.at[pl.ds(wid * tpw + i * tpc, tpc)])
        return c
    lax.fori_loop(0, tpw // tpc, chunk, 0)


@jax.jit
def _sc_combine(y, destf, topwf):
    mesh = plsc.VectorSubcoreMesh(core_axis_name="c", subcore_axis_name="s")
    kern = pl.kernel(
        _combine_body,
        out_type=jax.ShapeDtypeStruct((S, D), jnp.float32),
        mesh=mesh,
        compiler_params=pltpu.CompilerParams(needs_layout_passes=False),
        scratch_types=[
            pltpu.VMEM((LANES,), jnp.int32),
            pltpu.VMEM((LANES,), jnp.float32),
            pltpu.VMEM((LANES, D), jnp.float32),
            pltpu.VMEM((LANES // K, D), jnp.float32),
            pltpu.SemaphoreType.DMA,
        ],
    )
    return kern(y, destf, topwf)


def kernel(x, gate_w, w1, w2, w3):
    b, s, d = x.shape
    x2 = x.reshape(s, d)
    dest, topw, meta = _router(x2, gate_w)
    destf = dest.reshape(-1)
    topwf = topw.reshape(-1)
    bexp = meta[0]
    bval = meta[1]
    xg = _sc_gather(x2, destf)
    h = _ffn_in(xg, w1, w3, bexp, bval)
    y = _ffn_out(h, w2, bexp, bval)
    out = _sc_combine(y, destf, topwf)
    return out.reshape(b, s, d)


# speedup vs baseline: 1.4510x; 1.0197x over previous
import functools

import jax
import jax.numpy as jnp
from jax import lax
from jax.experimental import pallas as pl
from jax.experimental.pallas import tpu as pltpu
from jax.experimental.pallas import tpu_sc as plsc

S = 2048
D = 2048
E = 8
K = 2
F = 4096
P = S * K
TM = 256
NB = P // TM + E
NPAD = NB * TM
NW = 32
LANES = 16


def _router_body(x_ref, gw_ref, dest_ref, topw_ref, meta_ref):
    xv = x_ref[...]
    logits = jnp.dot(xv, gw_ref[...].T, preferred_element_type=jnp.float32)
    m = jnp.max(logits, axis=1, keepdims=True)
    p = jnp.exp(logits - m)
    p = p / jnp.sum(p, axis=1, keepdims=True)
    iota8 = lax.broadcasted_iota(jnp.int32, (S, E), 1)
    m1 = jnp.max(p, axis=1, keepdims=True)
    i1 = jnp.min(jnp.where(p >= m1, iota8, E), axis=1, keepdims=True)
    p2 = jnp.where(iota8 == i1, -1.0, p)
    m2 = jnp.max(p2, axis=1, keepdims=True)
    i2 = jnp.min(jnp.where(p2 >= m2, iota8, E), axis=1, keepdims=True)

    sel = jnp.logical_or(iota8 == i1, iota8 == i2).astype(jnp.float32)
    r_io = lax.broadcasted_iota(jnp.int32, (S, S), 0)
    c_io = lax.broadcasted_iota(jnp.int32, (S, S), 1)
    tril = (c_io < r_io).astype(jnp.float32)
    rank = jnp.dot(tril, sel, preferred_element_type=jnp.float32)
    cnt_f = rank[S - 1:S, :] + sel[S - 1:S, :]
    cnt = cnt_f.astype(jnp.int32)
    pc = ((cnt + (TM - 1)) // TM) * TM
    u8r = lax.broadcasted_iota(jnp.int32, (E, E), 0)
    u8c = lax.broadcasted_iota(jnp.int32, (E, E), 1)
    ltmask = (u8r < u8c).astype(jnp.float32)
    pstart_f = jnp.dot(pc.astype(jnp.float32), ltmask,
                       preferred_element_type=jnp.float32)
    pstart = pstart_f.astype(jnp.int32)

    base = rank.astype(jnp.int32) + pstart
    d1 = jnp.sum(jnp.where(iota8 == i1, base, 0), axis=1, keepdims=True)
    d2 = jnp.sum(jnp.where(iota8 == i2, base, 0), axis=1, keepdims=True)
    dest_ref[:, 0:1] = d1
    dest_ref[:, 1:2] = d2
    topw_ref[:, 0:1] = m1
    topw_ref[:, 1:2] = m2

    total = pstart[0:1, E - 1:E] + pc[0:1, E - 1:E]
    emax = jnp.max(jnp.where(cnt > 0, iota8[0:1, :], 0), axis=1,
                   keepdims=True)
    bb = lax.broadcasted_iota(jnp.int32, (1, 128), 1) * TM
    bexp = jnp.zeros((1, 128), jnp.int32)
    for ei in range(E):
        ps_e = pstart[0:1, ei:ei + 1]
        bexp = bexp + (bb >= ps_e).astype(jnp.int32)
    bexp = jnp.minimum(jnp.maximum(bexp - 1, 0), emax)
    bvalid = (bb < total).astype(jnp.int32)
    meta_ref[0:1, :] = bexp
    meta_ref[1:2, :] = bvalid


@jax.jit
def _router(x2, gate_w):
    return pl.pallas_call(
        _router_body,
        out_shape=[
            jax.ShapeDtypeStruct((S, K), jnp.int32),
            jax.ShapeDtypeStruct((S, K), jnp.float32),
            jax.ShapeDtypeStruct((2, 128), jnp.int32),
        ],
    )(x2, gate_w)


def _gather_body(x_ref, dest_ref, xg_ref, dloc, table, rows, rows2,
                 sem, sem2, wsem, wsem2):
    wid = lax.axis_index("s") * 2 + lax.axis_index("c")
    pltpu.sync_copy(dest_ref, dloc)

    def zero_blk(i, c):
        table[pl.ds(i * LANES, LANES)] = jnp.zeros((LANES,), jnp.int32)
        return c
    lax.fori_loop(0, NPAD // LANES, zero_blk, 0)

    def scat_blk(i, c):
        idx = dloc[pl.ds(i * LANES, LANES)]
        tok = (lax.iota(jnp.int32, LANES) + i * LANES) >> 1
        plsc.store_scatter(table, [idx], tok)
        return c
    lax.fori_loop(0, P // LANES, scat_blk, 0)

    spw = NPAD // NW
    base = wid * spw
    nch = spw // LANES
    bufs = (rows, rows2)
    gsems = (sem, sem2)
    wsems = (wsem, wsem2)
    hg = [None] * nch
    hw = [None] * nch
    for g in range(nch):
        if g >= 2:
            hw[g - 2].wait()
        sb = base + g * LANES
        hg[g] = pltpu.async_copy(
            x_ref.at[table.at[pl.ds(sb, LANES)]], bufs[g % 2], gsems[g % 2])
        if g >= 1:
            hg[g - 1].wait()
            hw[g - 1] = pltpu.async_copy(
                bufs[(g - 1) % 2], xg_ref.at[pl.ds(base + (g - 1) * LANES, LANES)],
                wsems[(g - 1) % 2])
    hg[nch - 1].wait()
    hw[nch - 1] = pltpu.async_copy(
        bufs[(nch - 1) % 2], xg_ref.at[pl.ds(base + (nch - 1) * LANES, LANES)],
        wsems[(nch - 1) % 2])
    hw[nch - 2].wait()
    hw[nch - 1].wait()


@jax.jit
def _sc_gather(x2, destf):
    mesh = plsc.VectorSubcoreMesh(core_axis_name="c", subcore_axis_name="s")
    kern = pl.kernel(
        _gather_body,
        out_type=jax.ShapeDtypeStruct((NPAD, D), jnp.float32),
        mesh=mesh,
        compiler_params=pltpu.CompilerParams(needs_layout_passes=False),
        scratch_types=[
            pltpu.VMEM((P,), jnp.int32),
            pltpu.VMEM((NPAD,), jnp.int32),
            pltpu.VMEM((LANES, D), jnp.float32),
            pltpu.VMEM((LANES, D), jnp.float32),
            pltpu.SemaphoreType.DMA,
            pltpu.SemaphoreType.DMA,
            pltpu.SemaphoreType.DMA,
            pltpu.SemaphoreType.DMA,
        ],
    )
    return kern(x2, destf)


FT1 = 1024
NF1 = F // FT1


def _ffn_in_body(bexp_s, bval_s, xg_ref, w1_ref, w3_ref, h_ref):
    b = pl.program_id(1)

    @pl.when(bval_s[b] == 1)
    def _():
        xv = xg_ref[...]
        a = jnp.dot(xv, w1_ref[0].T, preferred_element_type=jnp.float32)
        g = jnp.dot(xv, w3_ref[0].T, preferred_element_type=jnp.float32)
        h_ref[...] = ((a * jax.nn.sigmoid(a)) * g).astype(jnp.bfloat16)


@jax.jit
def _ffn_in(xg, w1, w3, bexp, bval):
    grid_spec = pltpu.PrefetchScalarGridSpec(
        num_scalar_prefetch=2,
        grid=(NF1, NB),
        in_specs=[
            pl.BlockSpec((TM, D), lambda f, b, be, bv: (b, 0)),
            pl.BlockSpec((1, FT1, D), lambda f, b, be, bv: (be[b], f, 0)),
            pl.BlockSpec((1, FT1, D), lambda f, b, be, bv: (be[b], f, 0)),
        ],
        out_specs=pl.BlockSpec((TM, FT1), lambda f, b, be, bv: (b, f)),
    )
    return pl.pallas_call(
        _ffn_in_body,
        grid_spec=grid_spec,
        out_shape=jax.ShapeDtypeStruct((NPAD, F), jnp.bfloat16),
        compiler_params=pltpu.CompilerParams(
            dimension_semantics=("arbitrary", "arbitrary")),
    )(bexp, bval, xg, w1, w3)


FT2 = 2048
NF2 = F // FT2


def _ffn_out_body(bexp_s, bval_s, h_ref, w2_ref, y_ref):
    b = pl.program_id(0)
    f = pl.program_id(1)

    @pl.when(bval_s[b] == 1)
    def _():
        w2c = w2_ref[0].astype(jnp.bfloat16)
        y = jnp.dot(h_ref[...], w2c.T, preferred_element_type=jnp.float32)

        @pl.when(f == 0)
        def _():
            y_ref[...] = y

        @pl.when(f > 0)
        def _():
            y_ref[...] += y


@jax.jit
def _ffn_out(h, w2, bexp, bval):
    def w2_map(b, f, be, bv):
        fs = jnp.where(b % 2 == 0, f, NF2 - 1 - f)
        return (be[b], 0, fs)

    def h_map(b, f, be, bv):
        fs = jnp.where(b % 2 == 0, f, NF2 - 1 - f)
        return (b, fs)

    grid_spec = pltpu.PrefetchScalarGridSpec(
        num_scalar_prefetch=2,
        grid=(NB, NF2),
        in_specs=[
            pl.BlockSpec((TM, FT2), h_map),
            pl.BlockSpec((1, D, FT2), w2_map),
        ],
        out_specs=pl.BlockSpec((TM, D), lambda b, f, be, bv: (b, 0)),
    )
    return pl.pallas_call(
        _ffn_out_body,
        grid_spec=grid_spec,
        out_shape=jax.ShapeDtypeStruct((NPAD, D), jnp.float32),
        compiler_params=pltpu.CompilerParams(
            dimension_semantics=("arbitrary", "arbitrary")),
    )(bexp, bval, h, w2)


def _combine_body(y_ref, dest_ref, w_ref, out_ref, didx, wloc, rows, acc, sem):
    wid = lax.axis_index("s") * 2 + lax.axis_index("c")
    tpw = S // NW
    tpc = LANES // K

    def chunk(i, c):
        pb = wid * tpw * K + i * LANES
        pltpu.sync_copy(dest_ref.at[pl.ds(pb, LANES)], didx)
        pltpu.async_copy(y_ref.at[didx], rows, sem).wait()
        pltpu.sync_copy(w_ref.at[pl.ds(pb, LANES)], wloc)
        wv = wloc[...]
        wj = [wv[j] for j in range(LANES)]

        def col(cc, c2):
            sl = pl.ds(cc * LANES, LANES)
            for j in range(tpc):
                acc[j, sl] = rows[2 * j, sl] * wj[2 * j] \
                    + rows[2 * j + 1, sl] * wj[2 * j + 1]
            return c2
        lax.fori_loop(0, D // LANES, col, 0)
        pltpu.sync_copy(acc, out_ref.at[pl.ds(wid * tpw + i * tpc, tpc)])
        return c
    lax.fori_loop(0, tpw // tpc, chunk, 0)


@jax.jit
def _sc_combine(y, destf, topwf):
    mesh = plsc.VectorSubcoreMesh(core_axis_name="c", subcore_axis_name="s")
    kern = pl.kernel(
        _combine_body,
        out_type=jax.ShapeDtypeStruct((S, D), jnp.float32),
        mesh=mesh,
        compiler_params=pltpu.CompilerParams(needs_layout_passes=False),
        scratch_types=[
            pltpu.VMEM((LANES,), jnp.int32),
            pltpu.VMEM((LANES,), jnp.float32),
            pltpu.VMEM((LANES, D), jnp.float32),
            pltpu.VMEM((LANES // K, D), jnp.float32),
            pltpu.SemaphoreType.DMA,
        ],
    )
    return kern(y, destf, topwf)


def kernel(x, gate_w, w1, w2, w3):
    b, s, d = x.shape
    x2 = x.reshape(s, d)
    dest, topw, meta = _router(x2, gate_w)
    destf = dest.reshape(-1)
    topwf = topw.reshape(-1)
    bexp = meta[0]
    bval = meta[1]
    xg = _sc_gather(x2, destf)
    h = _ffn_in(xg, w1, w3, bexp, bval)
    y = _ffn_out(h, w2, bexp, bval)
    out = _sc_combine(y, destf, topwf)
    return out.reshape(b, s, d)


# speedup vs baseline: 1.6527x; 1.1390x over previous
import functools

import jax
import jax.numpy as jnp
from jax import lax
from jax.experimental import pallas as pl
from jax.experimental.pallas import tpu as pltpu
from jax.experimental.pallas import tpu_sc as plsc

S = 2048
D = 2048
E = 8
K = 2
F = 4096
P = S * K
TM = 256
NB = P // TM + E
NPAD = NB * TM
NW = 32
LANES = 16


def _router_body(x_ref, gw_ref, dest_ref, topw_ref, meta_ref):
    xv = x_ref[...]
    logits = jnp.dot(xv, gw_ref[...].T, preferred_element_type=jnp.float32)
    m = jnp.max(logits, axis=1, keepdims=True)
    p = jnp.exp(logits - m)
    p = p / jnp.sum(p, axis=1, keepdims=True)
    iota8 = lax.broadcasted_iota(jnp.int32, (S, E), 1)
    m1 = jnp.max(p, axis=1, keepdims=True)
    i1 = jnp.min(jnp.where(p >= m1, iota8, E), axis=1, keepdims=True)
    p2 = jnp.where(iota8 == i1, -1.0, p)
    m2 = jnp.max(p2, axis=1, keepdims=True)
    i2 = jnp.min(jnp.where(p2 >= m2, iota8, E), axis=1, keepdims=True)

    sel = jnp.logical_or(iota8 == i1, iota8 == i2).astype(jnp.float32)
    r_io = lax.broadcasted_iota(jnp.int32, (S, S), 0)
    c_io = lax.broadcasted_iota(jnp.int32, (S, S), 1)
    tril = (c_io < r_io).astype(jnp.float32)
    rank = jnp.dot(tril, sel, preferred_element_type=jnp.float32)
    cnt_f = rank[S - 1:S, :] + sel[S - 1:S, :]
    cnt = cnt_f.astype(jnp.int32)
    pc = ((cnt + (TM - 1)) // TM) * TM
    u8r = lax.broadcasted_iota(jnp.int32, (E, E), 0)
    u8c = lax.broadcasted_iota(jnp.int32, (E, E), 1)
    ltmask = (u8r < u8c).astype(jnp.float32)
    pstart_f = jnp.dot(pc.astype(jnp.float32), ltmask,
                       preferred_element_type=jnp.float32)
    pstart = pstart_f.astype(jnp.int32)

    base = rank.astype(jnp.int32) + pstart
    d1 = jnp.sum(jnp.where(iota8 == i1, base, 0), axis=1, keepdims=True)
    d2 = jnp.sum(jnp.where(iota8 == i2, base, 0), axis=1, keepdims=True)
    dest_ref[:, 0:1] = d1
    dest_ref[:, 1:2] = d2
    topw_ref[:, 0:1] = m1
    topw_ref[:, 1:2] = m2

    total = pstart[0:1, E - 1:E] + pc[0:1, E - 1:E]
    emax = jnp.max(jnp.where(cnt > 0, iota8[0:1, :], 0), axis=1,
                   keepdims=True)
    bb = lax.broadcasted_iota(jnp.int32, (1, 128), 1) * TM
    bexp = jnp.zeros((1, 128), jnp.int32)
    for ei in range(E):
        ps_e = pstart[0:1, ei:ei + 1]
        bexp = bexp + (bb >= ps_e).astype(jnp.int32)
    bexp = jnp.minimum(jnp.maximum(bexp - 1, 0), emax)
    bvalid = (bb < total).astype(jnp.int32)
    meta_ref[0:1, :] = bexp
    meta_ref[1:2, :] = bvalid


@jax.jit
def _router(x2, gate_w):
    return pl.pallas_call(
        _router_body,
        out_shape=[
            jax.ShapeDtypeStruct((S, K), jnp.int32),
            jax.ShapeDtypeStruct((S, K), jnp.float32),
            jax.ShapeDtypeStruct((2, 128), jnp.int32),
        ],
    )(x2, gate_w)


def _gather_body(destt_ref, x_ref, xg_ref):
    b = pl.program_id(0)
    slot = lax.broadcasted_iota(jnp.int32, (TM, S), 0) + b * TM
    d0 = destt_ref[0:1, :]
    d1 = destt_ref[1:2, :]
    g = jnp.logical_or(d0 == slot, d1 == slot).astype(jnp.float32)
    xg_ref[...] = jnp.dot(g, x_ref[...],
                          preferred_element_type=jnp.float32).astype(jnp.bfloat16)


@jax.jit
def _tc_gather(x2, destt):
    return pl.pallas_call(
        _gather_body,
        grid=(NB,),
        in_specs=[
            pl.BlockSpec((K, S), lambda b: (0, 0)),
            pl.BlockSpec((S, D), lambda b: (0, 0)),
        ],
        out_specs=pl.BlockSpec((TM, D), lambda b: (b, 0)),
        out_shape=jax.ShapeDtypeStruct((NPAD, D), jnp.bfloat16),
        compiler_params=pltpu.CompilerParams(
            dimension_semantics=("arbitrary",)),
    )(destt, x2)


FT1 = 1024
NF1 = F // FT1


def _ffn_in_body(bexp_s, bval_s, xg_ref, w1_ref, w3_ref, h_ref):
    b = pl.program_id(1)

    @pl.when(bval_s[b] == 1)
    def _():
        xv = xg_ref[...].astype(jnp.float32)
        a = jnp.dot(xv, w1_ref[0].T, preferred_element_type=jnp.float32)
        g = jnp.dot(xv, w3_ref[0].T, preferred_element_type=jnp.float32)
        h_ref[...] = ((a * jax.nn.sigmoid(a)) * g).astype(jnp.bfloat16)


@jax.jit
def _ffn_in(xg, w1, w3, bexp, bval):
    grid_spec = pltpu.PrefetchScalarGridSpec(
        num_scalar_prefetch=2,
        grid=(NF1, NB),
        in_specs=[
            pl.BlockSpec((TM, D), lambda f, b, be, bv: (b, 0)),
            pl.BlockSpec((1, FT1, D), lambda f, b, be, bv: (be[b], f, 0)),
            pl.BlockSpec((1, FT1, D), lambda f, b, be, bv: (be[b], f, 0)),
        ],
        out_specs=pl.BlockSpec((TM, FT1), lambda f, b, be, bv: (b, f)),
    )
    return pl.pallas_call(
        _ffn_in_body,
        grid_spec=grid_spec,
        out_shape=jax.ShapeDtypeStruct((NPAD, F), jnp.bfloat16),
        compiler_params=pltpu.CompilerParams(
            dimension_semantics=("arbitrary", "arbitrary")),
    )(bexp, bval, xg, w1, w3)


FT2 = 2048
NF2 = F // FT2


def _ffn_out_body(bexp_s, bval_s, h_ref, w2_ref, y_ref):
    b = pl.program_id(0)
    f = pl.program_id(1)

    @pl.when(bval_s[b] == 1)
    def _():
        hv = h_ref[...].astype(jnp.float32)
        y = jnp.dot(hv, w2_ref[0].T, preferred_element_type=jnp.float32)

        @pl.when(f == 0)
        def _():
            y_ref[...] = y

        @pl.when(f > 0)
        def _():
            y_ref[...] += y


@jax.jit
def _ffn_out(h, w2, bexp, bval):
    def w2_map(b, f, be, bv):
        fs = jnp.where(b % 2 == 0, f, NF2 - 1 - f)
        return (be[b], 0, fs)

    def h_map(b, f, be, bv):
        fs = jnp.where(b % 2 == 0, f, NF2 - 1 - f)
        return (b, fs)

    grid_spec = pltpu.PrefetchScalarGridSpec(
        num_scalar_prefetch=2,
        grid=(NB, NF2),
        in_specs=[
            pl.BlockSpec((TM, FT2), h_map),
            pl.BlockSpec((1, D, FT2), w2_map),
        ],
        out_specs=pl.BlockSpec((TM, D), lambda b, f, be, bv: (b, 0)),
    )
    return pl.pallas_call(
        _ffn_out_body,
        grid_spec=grid_spec,
        out_shape=jax.ShapeDtypeStruct((NPAD, D), jnp.float32),
        compiler_params=pltpu.CompilerParams(
            dimension_semantics=("arbitrary", "arbitrary")),
    )(bexp, bval, h, w2)


def _combine_body(y_ref, dest_ref, w_ref, out_ref, didx, wloc, rows, acc, sem):
    wid = lax.axis_index("s") * 2 + lax.axis_index("c")
    tpw = S // NW
    tpc = LANES // K

    def chunk(i, c):
        pb = wid * tpw * K + i * LANES
        pltpu.sync_copy(dest_ref.at[pl.ds(pb, LANES)], didx)
        pltpu.async_copy(y_ref.at[didx], rows, sem).wait()
        pltpu.sync_copy(w_ref.at[pl.ds(pb, LANES)], wloc)
        wv = wloc[...]
        wj = [wv[j] for j in range(LANES)]

        def col(cc, c2):
            sl = pl.ds(cc * LANES, LANES)
            for j in range(tpc):
                acc[j, sl] = rows[2 * j, sl] * wj[2 * j] \
                    + rows[2 * j + 1, sl] * wj[2 * j + 1]
            return c2
        lax.fori_loop(0, D // LANES, col, 0)
        pltpu.sync_copy(acc, out_ref.at[pl.ds(wid * tpw + i * tpc, tpc)])
        return c
    lax.fori_loop(0, tpw // tpc, chunk, 0)


@jax.jit
def _sc_combine(y, destf, topwf):
    mesh = plsc.VectorSubcoreMesh(core_axis_name="c", subcore_axis_name="s")
    kern = pl.kernel(
        _combine_body,
        out_type=jax.ShapeDtypeStruct((S, D), jnp.float32),
        mesh=mesh,
        compiler_params=pltpu.CompilerParams(needs_layout_passes=False),
        scratch_types=[
            pltpu.VMEM((LANES,), jnp.int32),
            pltpu.VMEM((LANES,), jnp.float32),
            pltpu.VMEM((LANES, D), jnp.float32),
            pltpu.VMEM((LANES // K, D), jnp.float32),
            pltpu.SemaphoreType.DMA,
        ],
    )
    return kern(y, destf, topwf)


def kernel(x, gate_w, w1, w2, w3):
    b, s, d = x.shape
    x2 = x.reshape(s, d)
    dest, topw, meta = _router(x2, gate_w)
    destf = dest.reshape(-1)
    topwf = topw.reshape(-1)
    destt = dest.T
    bexp = meta[0]
    bval = meta[1]
    xg = _tc_gather(x2, destt)
    h = _ffn_in(xg, w1, w3, bexp, bval)
    y = _ffn_out(h, w2, bexp, bval)
    out = _sc_combine(y, destf, topwf)
    return out.reshape(b, s, d)


# speedup vs baseline: 1.7626x; 1.0665x over previous
import functools

import jax
import jax.numpy as jnp
from jax import lax
from jax.experimental import pallas as pl
from jax.experimental.pallas import tpu as pltpu
from jax.experimental.pallas import tpu_sc as plsc

S = 2048
D = 2048
E = 8
K = 2
F = 4096
P = S * K
TM = 256
NB = P // TM + E
NPAD = NB * TM
NW = 32
LANES = 16


def _router_body(x_ref, gw_ref, dest_ref, topw_ref, meta_ref):
    xv = x_ref[...]
    logits = jnp.dot(xv, gw_ref[...].T, preferred_element_type=jnp.float32)
    m = jnp.max(logits, axis=1, keepdims=True)
    p = jnp.exp(logits - m)
    p = p / jnp.sum(p, axis=1, keepdims=True)
    iota8 = lax.broadcasted_iota(jnp.int32, (S, E), 1)
    m1 = jnp.max(p, axis=1, keepdims=True)
    i1 = jnp.min(jnp.where(p >= m1, iota8, E), axis=1, keepdims=True)
    p2 = jnp.where(iota8 == i1, -1.0, p)
    m2 = jnp.max(p2, axis=1, keepdims=True)
    i2 = jnp.min(jnp.where(p2 >= m2, iota8, E), axis=1, keepdims=True)

    sel = jnp.logical_or(iota8 == i1, iota8 == i2).astype(jnp.float32)
    r_io = lax.broadcasted_iota(jnp.int32, (S, S), 0)
    c_io = lax.broadcasted_iota(jnp.int32, (S, S), 1)
    tril = (c_io < r_io).astype(jnp.float32)
    rank = jnp.dot(tril, sel, preferred_element_type=jnp.float32)
    cnt_f = rank[S - 1:S, :] + sel[S - 1:S, :]
    cnt = cnt_f.astype(jnp.int32)
    pc = ((cnt + (TM - 1)) // TM) * TM
    u8r = lax.broadcasted_iota(jnp.int32, (E, E), 0)
    u8c = lax.broadcasted_iota(jnp.int32, (E, E), 1)
    ltmask = (u8r < u8c).astype(jnp.float32)
    pstart_f = jnp.dot(pc.astype(jnp.float32), ltmask,
                       preferred_element_type=jnp.float32)
    pstart = pstart_f.astype(jnp.int32)

    base = rank.astype(jnp.int32) + pstart
    d1 = jnp.sum(jnp.where(iota8 == i1, base, 0), axis=1, keepdims=True)
    d2 = jnp.sum(jnp.where(iota8 == i2, base, 0), axis=1, keepdims=True)
    dest_ref[:, 0:1] = d1
    dest_ref[:, 1:2] = d2
    topw_ref[:, 0:1] = m1
    topw_ref[:, 1:2] = m2

    total = pstart[0:1, E - 1:E] + pc[0:1, E - 1:E]
    emax = jnp.max(jnp.where(cnt > 0, iota8[0:1, :], 0), axis=1,
                   keepdims=True)
    bb = lax.broadcasted_iota(jnp.int32, (1, 128), 1) * TM
    bexp = jnp.zeros((1, 128), jnp.int32)
    for ei in range(E):
        ps_e = pstart[0:1, ei:ei + 1]
        bexp = bexp + (bb >= ps_e).astype(jnp.int32)
    bexp = jnp.minimum(jnp.maximum(bexp - 1, 0), emax)
    bvalid = (bb < total).astype(jnp.int32)
    meta_ref[0:1, :] = bexp
    meta_ref[1:2, :] = bvalid


@jax.jit
def _router(x2, gate_w):
    return pl.pallas_call(
        _router_body,
        out_shape=[
            jax.ShapeDtypeStruct((S, K), jnp.int32),
            jax.ShapeDtypeStruct((S, K), jnp.float32),
            jax.ShapeDtypeStruct((2, 128), jnp.int32),
        ],
    )(x2, gate_w)


def _gather_body(destt_ref, x_ref, xg_ref):
    b = pl.program_id(0)
    slot = lax.broadcasted_iota(jnp.int32, (TM, S), 0) + b * TM
    d0 = destt_ref[0:1, :]
    d1 = destt_ref[1:2, :]
    g = jnp.logical_or(d0 == slot, d1 == slot).astype(jnp.float32)
    xg_ref[...] = jnp.dot(g, x_ref[...],
                          preferred_element_type=jnp.float32).astype(jnp.bfloat16)


@jax.jit
def _tc_gather(x2, destt):
    return pl.pallas_call(
        _gather_body,
        grid=(NB,),
        in_specs=[
            pl.BlockSpec((K, S), lambda b: (0, 0)),
            pl.BlockSpec((S, D), lambda b: (0, 0)),
        ],
        out_specs=pl.BlockSpec((TM, D), lambda b: (b, 0)),
        out_shape=jax.ShapeDtypeStruct((NPAD, D), jnp.bfloat16),
        compiler_params=pltpu.CompilerParams(
            dimension_semantics=("arbitrary",)),
    )(destt, x2)


FT1 = 1024
NF1 = F // FT1


def _ffn_in_body(bexp_s, bval_s, xg_ref, w1_ref, w3_ref, h_ref):
    b = pl.program_id(1)

    @pl.when(bval_s[b] == 1)
    def _():
        xv = xg_ref[...].astype(jnp.float32)
        a = jnp.dot(xv, w1_ref[0].T, preferred_element_type=jnp.float32)
        g = jnp.dot(xv, w3_ref[0].T, preferred_element_type=jnp.float32)
        h_ref[...] = ((a * jax.nn.sigmoid(a)) * g).astype(jnp.bfloat16)


@jax.jit
def _ffn_in(xg, w1, w3, bexp, bval):
    grid_spec = pltpu.PrefetchScalarGridSpec(
        num_scalar_prefetch=2,
        grid=(NF1, NB),
        in_specs=[
            pl.BlockSpec((TM, D), lambda f, b, be, bv: (b, 0)),
            pl.BlockSpec((1, FT1, D), lambda f, b, be, bv: (be[b], f, 0)),
            pl.BlockSpec((1, FT1, D), lambda f, b, be, bv: (be[b], f, 0)),
        ],
        out_specs=pl.BlockSpec((TM, FT1), lambda f, b, be, bv: (b, f)),
    )
    return pl.pallas_call(
        _ffn_in_body,
        grid_spec=grid_spec,
        out_shape=jax.ShapeDtypeStruct((NPAD, F), jnp.bfloat16),
        compiler_params=pltpu.CompilerParams(
            dimension_semantics=("arbitrary", "arbitrary")),
    )(bexp, bval, xg, w1, w3)


DT2 = 1024
ND2 = D // DT2


def _ffn_out_body(bexp_s, bval_s, h_ref, w2_ref, y_ref):
    b = pl.program_id(1)

    @pl.when(bval_s[b] == 1)
    def _():
        hv = h_ref[...].astype(jnp.float32)
        y_ref[...] = jnp.dot(hv, w2_ref[0].T,
                             preferred_element_type=jnp.float32)


@jax.jit
def _ffn_out(h, w2, bexp, bval):
    grid_spec = pltpu.PrefetchScalarGridSpec(
        num_scalar_prefetch=2,
        grid=(ND2, NB),
        in_specs=[
            pl.BlockSpec((TM, F), lambda dh, b, be, bv: (b, 0)),
            pl.BlockSpec((1, DT2, F), lambda dh, b, be, bv: (be[b], dh, 0)),
        ],
        out_specs=pl.BlockSpec((TM, DT2), lambda dh, b, be, bv: (b, dh)),
    )
    return pl.pallas_call(
        _ffn_out_body,
        grid_spec=grid_spec,
        out_shape=jax.ShapeDtypeStruct((NPAD, D), jnp.float32),
        compiler_params=pltpu.CompilerParams(
            dimension_semantics=("arbitrary", "arbitrary")),
    )(bexp, bval, h, w2)


def _combine_body(y_ref, dest_ref, w_ref, out_ref, didx, wloc, rows, acc, sem):
    wid = lax.axis_index("s") * 2 + lax.axis_index("c")
    tpw = S // NW
    tpc = LANES // K

    def chunk(i, c):
        pb = wid * tpw * K + i * LANES
        pltpu.sync_copy(dest_ref.at[pl.ds(pb, LANES)], didx)
        pltpu.async_copy(y_ref.at[didx], rows, sem).wait()
        pltpu.sync_copy(w_ref.at[pl.ds(pb, LANES)], wloc)
        wv = wloc[...]
        wj = [wv[j] for j in range(LANES)]

        def col(cc, c2):
            sl = pl.ds(cc * LANES, LANES)
            for j in range(tpc):
                acc[j, sl] = rows[2 * j, sl] * wj[2 * j] \
                    + rows[2 * j + 1, sl] * wj[2 * j + 1]
            return c2
        lax.fori_loop(0, D // LANES, col, 0)
        pltpu.sync_copy(acc, out_ref.at[pl.ds(wid * tpw + i * tpc, tpc)])
        return c
    lax.fori_loop(0, tpw // tpc, chunk, 0)


@jax.jit
def _sc_combine(y, destf, topwf):
    mesh = plsc.VectorSubcoreMesh(core_axis_name="c", subcore_axis_name="s")
    kern = pl.kernel(
        _combine_body,
        out_type=jax.ShapeDtypeStruct((S, D), jnp.float32),
        mesh=mesh,
        compiler_params=pltpu.CompilerParams(needs_layout_passes=False),
        scratch_types=[
            pltpu.VMEM((LANES,), jnp.int32),
            pltpu.VMEM((LANES,), jnp.float32),
            pltpu.VMEM((LANES, D), jnp.float32),
            pltpu.VMEM((LANES // K, D), jnp.float32),
            pltpu.SemaphoreType.DMA,
        ],
    )
    return kern(y, destf, topwf)


def kernel(x, gate_w, w1, w2, w3):
    b, s, d = x.shape
    x2 = x.reshape(s, d)
    dest, topw, meta = _router(x2, gate_w)
    destf = dest.reshape(-1)
    topwf = topw.reshape(-1)
    destt = dest.T
    bexp = meta[0]
    bval = meta[1]
    xg = _tc_gather(x2, destt)
    h = _ffn_in(xg, w1, w3, bexp, bval)
    y = _ffn_out(h, w2, bexp, bval)
    out = _sc_combine(y, destf, topwf)
    return out.reshape(b, s, d)
